# Initial kernel scaffold; baseline (speedup 1.0000x reference)
import jax
import jax.numpy as jnp
from jax.experimental import pallas as pl


def kernel(x, edge_index, Wc, bc, gamma, beta, Wo, bo):
    raise NotImplementedError("write your pallas kernel here")


# speedup vs baseline: 5.0430x; 5.0430x over previous
import functools

import jax
import jax.numpy as jnp
from jax import lax
from jax.experimental import pallas as pl
from jax.experimental.pallas import tpu as pltpu
from jax.experimental.pallas import tpu_sc as plsc

N = 10000
E = 160000
NHID = 256
NOUT = 128
HALF = NHID // 2
L = 3
EPS = 1e-5

NC = 2
NS = 16
NACC = 10240
RPT = NACC // NS


DK = 40
DET2 = E // (NC * NS)
DCH2 = DET2 // DK
ZR = 128

@functools.cache
def _sc_mesh():
    return plsc.VectorSubcoreMesh(core_axis_name="c", subcore_axis_name="s")


def _deg_body(dst_hbm, deg_hbm, ones_v, zed_v, idx_v, acc_sh):
    c = lax.axis_index("c")
    s = lax.axis_index("s")
    wid = s * NC + c

    def fo(i, _):
        for j in range(HALF // 16):
            ones_v[i, pl.ds(j * 16, 16)] = jnp.ones((16,), jnp.float32)
        return 0
    lax.fori_loop(0, DK, fo, 0)

    def fz(i, _):
        for j in range(HALF // 16):
            zed_v[i, pl.ds(j * 16, 16)] = jnp.zeros((16,), jnp.float32)
        return 0
    lax.fori_loop(0, ZR, fz, 0)

    def zcp(i, _):
        pltpu.sync_copy(zed_v, acc_sh.at[pl.ds(s * RPT + i * ZR, ZR)])
        return 0
    lax.fori_loop(0, RPT // ZR, zcp, 0)

    plsc.subcore_barrier()

    def chunk(i, _):
        base = wid * DET2 + i * DK
        pltpu.sync_copy(dst_hbm.at[pl.ds(base, DK)], idx_v)
        pltpu.sync_copy(ones_v, acc_sh.at[idx_v], add=True)
        return 0
    lax.fori_loop(0, DCH2, chunk, 0)

    plsc.subcore_barrier()

    def outcp(i, _):
        pltpu.sync_copy(acc_sh.at[pl.ds(s * RPT + i * ZR, ZR)], zed_v)
        pltpu.sync_copy(zed_v, deg_hbm.at[c, pl.ds(s * RPT + i * ZR, ZR)])
        return 0
    lax.fori_loop(0, RPT // ZR, outcp, 0)


@functools.cache
def _deg_call():
    return pl.kernel(
        _deg_body,
        out_type=jax.ShapeDtypeStruct((NC, NACC, HALF), jnp.float32),
        mesh=_sc_mesh(),
        scratch_types=[
            pltpu.VMEM((DK, HALF), jnp.float32),
            pltpu.VMEM((ZR, HALF), jnp.float32),
            pltpu.VMEM((DK,), jnp.int32),
            pltpu.VMEM_SHARED((NACC, HALF), jnp.float32),
        ],
    )


AK = 80
AET = E // NS
ACH = AET // AK
ZR = 128


def _agg_body(hw2_hbm, src_hbm, dst_hbm, out_hbm,
              idxs_v, idxd_v, rows_v, zed_v, acc_sh, sem):
    c = lax.axis_index("c")
    s = lax.axis_index("s")

    def fz(i, _):
        for j in range(HALF // 16):
            zed_v[i, pl.ds(j * 16, 16)] = jnp.zeros((16,), jnp.float32)
        return 0
    lax.fori_loop(0, ZR, fz, 0)

    def zcp(i, _):
        pltpu.sync_copy(zed_v, acc_sh.at[pl.ds(s * RPT + i * ZR, ZR)])
        return 0
    lax.fori_loop(0, RPT // ZR, zcp, 0)
    plsc.subcore_barrier()

    coff = jnp.full((16,), c * N, jnp.int32)

    def chunk(i, _):
        base = s * AET + i * AK
        pltpu.sync_copy(src_hbm.at[pl.ds(base, AK)], idxs_v)
        pltpu.sync_copy(dst_hbm.at[pl.ds(base, AK)], idxd_v)
        for j in range(AK // 16):
            idxs_v[pl.ds(j * 16, 16)] = idxs_v[pl.ds(j * 16, 16)] + coff
        pltpu.async_copy(hw2_hbm.at[idxs_v], rows_v, sem).wait()
        pltpu.sync_copy(rows_v, acc_sh.at[idxd_v], add=True)
        return 0
    lax.fori_loop(0, ACH, chunk, 0)

    plsc.subcore_barrier()

    def outcp(i, _):
        pltpu.sync_copy(acc_sh.at[pl.ds(s * RPT + i * ZR, ZR)], zed_v)
        pltpu.sync_copy(zed_v, out_hbm.at[c, pl.ds(s * RPT + i * ZR, ZR)])
        return 0
    lax.fori_loop(0, RPT // ZR, outcp, 0)


@functools.cache
def _agg_call():
    return pl.kernel(
        _agg_body,
        out_type=jax.ShapeDtypeStruct((NC, NACC, HALF), jnp.float32),
        mesh=_sc_mesh(),
        scratch_types=[
            pltpu.VMEM((AK,), jnp.int32),
            pltpu.VMEM((AK,), jnp.int32),
            pltpu.VMEM((AK, HALF), jnp.float32),
            pltpu.VMEM((ZR, HALF), jnp.float32),
            pltpu.VMEM_SHARED((NACC, HALF), jnp.float32),
            pltpu.SemaphoreType.DMA,
        ],
    )


BM = 1000
NB = N // BM


def _dinv_of(deg_ref):
    deg = deg_ref[0, :, 0:1] + deg_ref[1, :, 0:1]
    return lax.rsqrt(jnp.maximum(deg, 1.0))


def _mm0_body(x_ref, deg_ref, w_ref, out_ref):
    dinv = _dinv_of(deg_ref)
    hw = jnp.dot(x_ref[...], w_ref[...], preferred_element_type=jnp.float32)
    hs = hw * dinv
    out_ref[0] = hs[:, :HALF]
    out_ref[1] = hs[:, HALF:]


def _mm0(x, deg16, w):
    return pl.pallas_call(
        _mm0_body,
        grid=(NB,),
        in_specs=[
            pl.BlockSpec((BM, NHID), lambda i: (i, 0)),
            pl.BlockSpec((NC, BM, HALF), lambda i: (0, i, 0)),
            pl.BlockSpec((NHID, NHID), lambda i: (0, 0)),
        ],
        out_specs=pl.BlockSpec((NC, BM, HALF), lambda i: (0, i, 0)),
        out_shape=jax.ShapeDtypeStruct((NC, N, HALF), jnp.float32),
    )(x, deg16, w)


def _stats_body(agg_ref, deg_ref, out_ref):
    i = pl.program_id(0)

    @pl.when(i == 0)
    def _():
        out_ref[...] = jnp.zeros_like(out_ref)

    dinv = _dinv_of(deg_ref)
    a = jnp.concatenate([agg_ref[0] * dinv, agg_ref[1] * dinv], axis=1)
    out_ref[0:1, :] += jnp.sum(a, axis=0, keepdims=True)
    out_ref[1:2, :] += jnp.sum(a * a, axis=0, keepdims=True)


def _stats(agg, deg16):
    return pl.pallas_call(
        _stats_body,
        grid=(NB,),
        in_specs=[
            pl.BlockSpec((NC, BM, HALF), lambda i: (0, i, 0)),
            pl.BlockSpec((NC, BM, HALF), lambda i: (0, i, 0)),
        ],
        out_specs=pl.BlockSpec((2, NHID), lambda i: (0, 0)),
        out_shape=jax.ShapeDtypeStruct((2, NHID), jnp.float32),
    )(agg, deg16)


def _bn_h(agg_ref, deg_ref, sums_ref, gam_ref, bet_ref):
    dinv = _dinv_of(deg_ref)
    a = jnp.concatenate([agg_ref[0] * dinv, agg_ref[1] * dinv], axis=1)
    mean = sums_ref[0:1, :] * (1.0 / N)
    var = sums_ref[1:2, :] * (1.0 / N) - mean * mean
    rstd = lax.rsqrt(var + EPS)
    h = jnp.maximum((a - mean) * (rstd * gam_ref[...]) + bet_ref[...], 0.0)
    return h, dinv


def _bnmm_mid_body(agg_ref, deg_ref, sums_ref, gam_ref, bet_ref, w_ref,
                   out_ref):
    h, dinv = _bn_h(agg_ref, deg_ref, sums_ref, gam_ref, bet_ref)
    hw = jnp.dot(h * dinv, w_ref[...], preferred_element_type=jnp.float32)
    out_ref[0] = hw[:, :HALF]
    out_ref[1] = hw[:, HALF:]


def _bnmm_mid(agg, deg16, sums, gam, bet, w):
    return pl.pallas_call(
        _bnmm_mid_body,
        grid=(NB,),
        in_specs=[
            pl.BlockSpec((NC, BM, HALF), lambda i: (0, i, 0)),
            pl.BlockSpec((NC, BM, HALF), lambda i: (0, i, 0)),
            pl.BlockSpec((2, NHID), lambda i: (0, 0)),
            pl.BlockSpec((1, NHID), lambda i: (0, 0)),
            pl.BlockSpec((1, NHID), lambda i: (0, 0)),
            pl.BlockSpec((NHID, NHID), lambda i: (0, 0)),
        ],
        out_specs=pl.BlockSpec((NC, BM, HALF), lambda i: (0, i, 0)),
        out_shape=jax.ShapeDtypeStruct((NC, N, HALF), jnp.float32),
    )(agg, deg16, sums, gam, bet, w)


def _bnmm_fin_body(agg_ref, deg_ref, sums_ref, gam_ref, bet_ref, w_ref,
                   b_ref, out_ref):
    h, _ = _bn_h(agg_ref, deg_ref, sums_ref, gam_ref, bet_ref)
    out_ref[...] = (
        jnp.dot(h, w_ref[...], preferred_element_type=jnp.float32)
        + b_ref[...]
    )


def _bnmm_fin(agg, deg16, sums, gam, bet, w, b):
    return pl.pallas_call(
        _bnmm_fin_body,
        grid=(NB,),
        in_specs=[
            pl.BlockSpec((NC, BM, HALF), lambda i: (0, i, 0)),
            pl.BlockSpec((NC, BM, HALF), lambda i: (0, i, 0)),
            pl.BlockSpec((2, NHID), lambda i: (0, 0)),
            pl.BlockSpec((1, NHID), lambda i: (0, 0)),
            pl.BlockSpec((1, NHID), lambda i: (0, 0)),
            pl.BlockSpec((NHID, NOUT), lambda i: (0, 0)),
            pl.BlockSpec((1, NOUT), lambda i: (0, 0)),
        ],
        out_specs=pl.BlockSpec((BM, NOUT), lambda i: (i, 0)),
        out_shape=jax.ShapeDtypeStruct((N, NOUT), jnp.float32),
    )(agg, deg16, sums, gam, bet, w, b)


def kernel(x, edge_index, Wc, bc, gamma, beta, Wo, bo):
    src = edge_index[0]
    dst = edge_index[1]
    deg16 = _deg_call()(dst)
    h2 = _mm0(x, deg16, Wc[0])
    out = None
    for l in range(L):
        agg = _agg_call()(h2.reshape(NC * N, HALF), src, dst)
        sums = _stats(agg, deg16)
        gam = gamma[l].reshape(1, NHID)
        bet = beta[l].reshape(1, NHID)
        if l < L - 1:
            h2 = _bnmm_mid(agg, deg16, sums, gam, bet, Wc[l + 1])
        else:
            out = _bnmm_fin(agg, deg16, sums, gam, bet, Wo,
                            bo.reshape(1, NOUT))
    return out


# speedup vs baseline: 5.5945x; 1.1093x over previous
import functools

import jax
import jax.numpy as jnp
from jax import lax
from jax.experimental import pallas as pl
from jax.experimental.pallas import tpu as pltpu
from jax.experimental.pallas import tpu_sc as plsc

N = 10000
E = 160000
NHID = 256
NOUT = 128
HALF = NHID // 2
L = 3
EPS = 1e-5

NC = 2
NS = 16
NACC = 10240
RPT = NACC // NS


EPAD = 163840
DEGW = 128
DK = 128
DET2 = EPAD // (NC * NS)
DCH2 = DET2 // DK
ZR = 128

@functools.cache
def _sc_mesh():
    return plsc.VectorSubcoreMesh(core_axis_name="c", subcore_axis_name="s")


def _deg_body(dst_hbm, deg_hbm, ones_v, zed_v, idx_v, acc_sh):
    c = lax.axis_index("c")
    s = lax.axis_index("s")
    wid = s * NC + c

    def fo(i, _):
        for j in range(DEGW // 16):
            ones_v[i, pl.ds(j * 16, 16)] = jnp.ones((16,), jnp.float32)
        return 0
    lax.fori_loop(0, DK, fo, 0)

    def fz(i, _):
        for j in range(DEGW // 16):
            zed_v[i, pl.ds(j * 16, 16)] = jnp.zeros((16,), jnp.float32)
        return 0
    lax.fori_loop(0, ZR, fz, 0)

    def zcp(i, _):
        pltpu.sync_copy(zed_v, acc_sh.at[pl.ds(s * RPT + i * ZR, ZR)])
        return 0
    lax.fori_loop(0, RPT // ZR, zcp, 0)

    plsc.subcore_barrier()

    def chunk(i, _):
        base = wid * DET2 + i * DK
        pltpu.sync_copy(dst_hbm.at[pl.ds(base, DK)], idx_v)
        pltpu.sync_copy(ones_v, acc_sh.at[idx_v], add=True)
        return 0
    lax.fori_loop(0, DCH2, chunk, 0)

    plsc.subcore_barrier()

    def outcp(i, _):
        pltpu.sync_copy(acc_sh.at[pl.ds(s * RPT + i * ZR, ZR)], zed_v)
        pltpu.sync_copy(zed_v, deg_hbm.at[c, pl.ds(s * RPT + i * ZR, ZR)])
        return 0
    lax.fori_loop(0, RPT // ZR, outcp, 0)


@functools.cache
def _deg_call():
    return pl.kernel(
        _deg_body,
        out_type=jax.ShapeDtypeStruct((NC, NACC, DEGW), jnp.float32),
        mesh=_sc_mesh(),
        scratch_types=[
            pltpu.VMEM((DK, DEGW), jnp.float32),
            pltpu.VMEM((ZR, DEGW), jnp.float32),
            pltpu.VMEM((DK,), jnp.int32),
            pltpu.VMEM_SHARED((NACC, DEGW), jnp.float32),
        ],
    )


AK = 128
AET = EPAD // NS
ACH = AET // AK


def _agg_body(hw2_hbm, src_hbm, dst_hbm, out_hbm,
              idxs_a, idxd_a, rows_a, idxs_b, idxd_b, rows_b,
              acc_sh, sem_a, sem_b):
    c = lax.axis_index("c")
    s = lax.axis_index("s")
    tbase = s * AET
    coff = jnp.full((16,), c * N, jnp.int32)

    def load_start(base, idxs_v, idxd_v, rows_v, sem):
        pltpu.sync_copy(src_hbm.at[pl.ds(base, AK)], idxs_v)
        pltpu.sync_copy(dst_hbm.at[pl.ds(base, AK)], idxd_v)
        for j in range(AK // 16):
            idxs_v[pl.ds(j * 16, 16)] = idxs_v[pl.ds(j * 16, 16)] + coff
        pltpu.async_copy(hw2_hbm.at[idxs_v], rows_v, sem)

    def fz(i, _):
        for j in range(HALF // 16):
            rows_a[i, pl.ds(j * 16, 16)] = jnp.zeros((16,), jnp.float32)
        return 0
    lax.fori_loop(0, ZR, fz, 0)

    def zcp(i, _):
        pltpu.sync_copy(rows_a, acc_sh.at[pl.ds(s * RPT + i * ZR, ZR)])
        return 0
    lax.fori_loop(0, RPT // ZR, zcp, 0)

    load_start(tbase, idxs_a, idxd_a, rows_a, sem_a)
    plsc.subcore_barrier()

    def pair(g, _):
        i0 = 2 * g
        load_start(tbase + (i0 + 1) * AK, idxs_b, idxd_b, rows_b, sem_b)
        pltpu.make_async_copy(hw2_hbm.at[idxs_a], rows_a, sem_a).wait()
        pltpu.sync_copy(rows_a, acc_sh.at[idxd_a], add=True)

        @pl.when(i0 + 2 < ACH)
        def _():
            load_start(tbase + (i0 + 2) * AK, idxs_a, idxd_a, rows_a, sem_a)

        pltpu.make_async_copy(hw2_hbm.at[idxs_b], rows_b, sem_b).wait()
        pltpu.sync_copy(rows_b, acc_sh.at[idxd_b], add=True)
        return 0
    lax.fori_loop(0, ACH // 2, pair, 0)

    plsc.subcore_barrier()

    def outcp(i, _):
        pltpu.sync_copy(acc_sh.at[pl.ds(s * RPT + i * ZR, ZR)], rows_a)
        pltpu.sync_copy(rows_a, out_hbm.at[c, pl.ds(s * RPT + i * ZR, ZR)])
        return 0
    lax.fori_loop(0, RPT // ZR, outcp, 0)


@functools.cache
def _agg_call():
    return pl.kernel(
        _agg_body,
        out_type=jax.ShapeDtypeStruct((NC, NACC, HALF), jnp.float32),
        mesh=_sc_mesh(),
        scratch_types=[
            pltpu.VMEM((AK,), jnp.int32),
            pltpu.VMEM((AK,), jnp.int32),
            pltpu.VMEM((AK, HALF), jnp.float32),
            pltpu.VMEM((AK,), jnp.int32),
            pltpu.VMEM((AK,), jnp.int32),
            pltpu.VMEM((AK, HALF), jnp.float32),
            pltpu.VMEM_SHARED((NACC, HALF), jnp.float32),
            pltpu.SemaphoreType.DMA,
            pltpu.SemaphoreType.DMA,
        ],
    )


BM = 1000
NB = N // BM


def _dinv_of(deg_ref):
    deg = deg_ref[0, :, 0:1] + deg_ref[1, :, 0:1]
    return lax.rsqrt(jnp.maximum(deg, 1.0))


def _mm0_body(x_ref, deg_ref, w_ref, out_ref):
    dinv = _dinv_of(deg_ref)
    hw = jnp.dot(x_ref[...], w_ref[...], preferred_element_type=jnp.float32)
    hs = hw * dinv
    out_ref[0] = hs[:, :HALF]
    out_ref[1] = hs[:, HALF:]


def _mm0(x, deg16, w):
    return pl.pallas_call(
        _mm0_body,
        grid=(NB,),
        in_specs=[
            pl.BlockSpec((BM, NHID), lambda i: (i, 0)),
            pl.BlockSpec((NC, BM, DEGW), lambda i: (0, i, 0)),
            pl.BlockSpec((NHID, NHID), lambda i: (0, 0)),
        ],
        out_specs=pl.BlockSpec((NC, BM, HALF), lambda i: (0, i, 0)),
        out_shape=jax.ShapeDtypeStruct((NC, N, HALF), jnp.float32),
    )(x, deg16, w)


def _stats_body(agg_ref, deg_ref, out_ref):
    i = pl.program_id(0)

    @pl.when(i == 0)
    def _():
        out_ref[...] = jnp.zeros_like(out_ref)

    dinv = _dinv_of(deg_ref)
    a = jnp.concatenate([agg_ref[0] * dinv, agg_ref[1] * dinv], axis=1)
    out_ref[0:1, :] += jnp.sum(a, axis=0, keepdims=True)
    out_ref[1:2, :] += jnp.sum(a * a, axis=0, keepdims=True)


def _stats(agg, deg16):
    return pl.pallas_call(
        _stats_body,
        grid=(NB,),
        in_specs=[
            pl.BlockSpec((NC, BM, HALF), lambda i: (0, i, 0)),
            pl.BlockSpec((NC, BM, DEGW), lambda i: (0, i, 0)),
        ],
        out_specs=pl.BlockSpec((2, NHID), lambda i: (0, 0)),
        out_shape=jax.ShapeDtypeStruct((2, NHID), jnp.float32),
    )(agg, deg16)


def _bn_h(agg_ref, deg_ref, sums_ref, gam_ref, bet_ref):
    dinv = _dinv_of(deg_ref)
    a = jnp.concatenate([agg_ref[0] * dinv, agg_ref[1] * dinv], axis=1)
    mean = sums_ref[0:1, :] * (1.0 / N)
    var = sums_ref[1:2, :] * (1.0 / N) - mean * mean
    rstd = lax.rsqrt(var + EPS)
    h = jnp.maximum((a - mean) * (rstd * gam_ref[...]) + bet_ref[...], 0.0)
    return h, dinv


def _bnmm_mid_body(agg_ref, deg_ref, sums_ref, gam_ref, bet_ref, w_ref,
                   out_ref):
    h, dinv = _bn_h(agg_ref, deg_ref, sums_ref, gam_ref, bet_ref)
    hw = jnp.dot(h * dinv, w_ref[...], preferred_element_type=jnp.float32)
    out_ref[0] = hw[:, :HALF]
    out_ref[1] = hw[:, HALF:]


def _bnmm_mid(agg, deg16, sums, gam, bet, w):
    return pl.pallas_call(
        _bnmm_mid_body,
        grid=(NB,),
        in_specs=[
            pl.BlockSpec((NC, BM, HALF), lambda i: (0, i, 0)),
            pl.BlockSpec((NC, BM, DEGW), lambda i: (0, i, 0)),
            pl.BlockSpec((2, NHID), lambda i: (0, 0)),
            pl.BlockSpec((1, NHID), lambda i: (0, 0)),
            pl.BlockSpec((1, NHID), lambda i: (0, 0)),
            pl.BlockSpec((NHID, NHID), lambda i: (0, 0)),
        ],
        out_specs=pl.BlockSpec((NC, BM, HALF), lambda i: (0, i, 0)),
        out_shape=jax.ShapeDtypeStruct((NC, N, HALF), jnp.float32),
    )(agg, deg16, sums, gam, bet, w)


def _bnmm_fin_body(agg_ref, deg_ref, sums_ref, gam_ref, bet_ref, w_ref,
                   b_ref, out_ref):
    h, _ = _bn_h(agg_ref, deg_ref, sums_ref, gam_ref, bet_ref)
    out_ref[...] = (
        jnp.dot(h, w_ref[...], preferred_element_type=jnp.float32)
        + b_ref[...]
    )


def _bnmm_fin(agg, deg16, sums, gam, bet, w, b):
    return pl.pallas_call(
        _bnmm_fin_body,
        grid=(NB,),
        in_specs=[
            pl.BlockSpec((NC, BM, HALF), lambda i: (0, i, 0)),
            pl.BlockSpec((NC, BM, DEGW), lambda i: (0, i, 0)),
            pl.BlockSpec((2, NHID), lambda i: (0, 0)),
            pl.BlockSpec((1, NHID), lambda i: (0, 0)),
            pl.BlockSpec((1, NHID), lambda i: (0, 0)),
            pl.BlockSpec((NHID, NOUT), lambda i: (0, 0)),
            pl.BlockSpec((1, NOUT), lambda i: (0, 0)),
        ],
        out_specs=pl.BlockSpec((BM, NOUT), lambda i: (i, 0)),
        out_shape=jax.ShapeDtypeStruct((N, NOUT), jnp.float32),
    )(agg, deg16, sums, gam, bet, w, b)


def kernel(x, edge_index, Wc, bc, gamma, beta, Wo, bo):
    src = edge_index[0]
    dst = edge_index[1]
    pad = EPAD - E
    srcp = jnp.concatenate([src, jnp.zeros((pad,), jnp.int32)])
    dstp = jnp.concatenate([dst, jnp.full((pad,), N, jnp.int32)])
    deg16 = _deg_call()(dstp)
    h2 = _mm0(x, deg16, Wc[0])
    out = None
    for l in range(L):
        agg = _agg_call()(h2.reshape(NC * N, HALF), srcp, dstp)
        sums = _stats(agg, deg16)
        gam = gamma[l].reshape(1, NHID)
        bet = beta[l].reshape(1, NHID)
        if l < L - 1:
            h2 = _bnmm_mid(agg, deg16, sums, gam, bet, Wc[l + 1])
        else:
            out = _bnmm_fin(agg, deg16, sums, gam, bet, Wo,
                            bo.reshape(1, NOUT))
    return out


# speedup vs baseline: 6.1096x; 1.0921x over previous
import functools

import jax
import jax.numpy as jnp
from jax import lax
from jax.experimental import pallas as pl
from jax.experimental.pallas import tpu as pltpu
from jax.experimental.pallas import tpu_sc as plsc

N = 10000
E = 160000
NHID = 256
NOUT = 128
HALF = NHID // 2
L = 3
EPS = 1e-5

NC = 2
NS = 16
NACC = 10240
RPT = NACC // NS


EPAD = 163840
DEGW = 128
DK = 128
DET2 = EPAD // (NC * NS)
DCH2 = DET2 // DK
ZR = 128

@functools.cache
def _sc_mesh():
    return plsc.VectorSubcoreMesh(core_axis_name="c", subcore_axis_name="s")


def _deg_body(dst_hbm, deg_hbm, ones_v, zed_v, idx_v, acc_sh):
    c = lax.axis_index("c")
    s = lax.axis_index("s")
    wid = s * NC + c

    def fo(i, _):
        for j in range(DEGW // 16):
            ones_v[i, pl.ds(j * 16, 16)] = jnp.ones((16,), jnp.float32)
        return 0
    lax.fori_loop(0, DK, fo, 0)

    def fz(i, _):
        for j in range(DEGW // 16):
            zed_v[i, pl.ds(j * 16, 16)] = jnp.zeros((16,), jnp.float32)
        return 0
    lax.fori_loop(0, ZR, fz, 0)

    def zcp(i, _):
        pltpu.sync_copy(zed_v, acc_sh.at[pl.ds(s * RPT + i * ZR, ZR)])
        return 0
    lax.fori_loop(0, RPT // ZR, zcp, 0)

    plsc.subcore_barrier()

    def chunk(i, _):
        base = wid * DET2 + i * DK
        pltpu.sync_copy(dst_hbm.at[pl.ds(base, DK)], idx_v)
        pltpu.sync_copy(ones_v, acc_sh.at[idx_v], add=True)
        return 0
    lax.fori_loop(0, DCH2, chunk, 0)

    plsc.subcore_barrier()

    def outcp(i, _):
        pltpu.sync_copy(acc_sh.at[pl.ds(s * RPT + i * ZR, ZR)], zed_v)
        pltpu.sync_copy(zed_v, deg_hbm.at[c, pl.ds(s * RPT + i * ZR, ZR)])
        return 0
    lax.fori_loop(0, RPT // ZR, outcp, 0)


@functools.cache
def _deg_call():
    return pl.kernel(
        _deg_body,
        out_type=jax.ShapeDtypeStruct((NC, NACC, DEGW), jnp.float32),
        mesh=_sc_mesh(),
        scratch_types=[
            pltpu.VMEM((DK, DEGW), jnp.float32),
            pltpu.VMEM((ZR, DEGW), jnp.float32),
            pltpu.VMEM((DK,), jnp.int32),
            pltpu.VMEM_SHARED((NACC, DEGW), jnp.float32),
        ],
    )


AK = 128
AET = EPAD // NS
ACH = AET // AK


def _agg_body(hw2_hbm, srcb_hbm, dst_hbm, out_hbm,
              idxs_v, idxd_v, rows_a, rows_b,
              sem_i0, sem_i1, sem_i2, sem_i3, acc_sh, sem_a, sem_b):
    c = lax.axis_index("c")
    s = lax.axis_index("s")
    sb = c * EPAD + s * AET
    db = s * AET
    sem_i = (sem_i0, sem_i1, sem_i2, sem_i3)

    def issue_idx(q, r):
        pltpu.async_copy(srcb_hbm.at[pl.ds(sb + q * AK, AK)],
                         idxs_v.at[r], sem_i[r])
        pltpu.async_copy(dst_hbm.at[pl.ds(db + q * AK, AK)],
                         idxd_v.at[r], sem_i[r])

    def wait_idx(q, r):
        pltpu.make_async_copy(srcb_hbm.at[pl.ds(sb + q * AK, AK)],
                              idxs_v.at[r], sem_i[r]).wait()
        pltpu.make_async_copy(dst_hbm.at[pl.ds(db + q * AK, AK)],
                              idxd_v.at[r], sem_i[r]).wait()

    def gather_start(r, rows_v, sem):
        pltpu.async_copy(hw2_hbm.at[idxs_v.at[r]], rows_v, sem)

    def finish(r, rows_v, sem):
        pltpu.make_async_copy(hw2_hbm.at[idxs_v.at[r]], rows_v, sem).wait()
        pltpu.sync_copy(rows_v, acc_sh.at[idxd_v.at[r]], add=True)

    for r in range(4):
        issue_idx(r, r)

    def fz(i, _):
        for j in range(HALF // 16):
            rows_a[i, pl.ds(j * 16, 16)] = jnp.zeros((16,), jnp.float32)
        return 0
    lax.fori_loop(0, ZR, fz, 0)

    def zcp(i, _):
        pltpu.sync_copy(rows_a, acc_sh.at[pl.ds(s * RPT + i * ZR, ZR)])
        return 0
    lax.fori_loop(0, RPT // ZR, zcp, 0)

    wait_idx(0, 0)
    gather_start(0, rows_a, sem_a)
    plsc.subcore_barrier()

    def quad(t, _):
        b = 4 * t
        wait_idx(b + 1, 1)
        gather_start(1, rows_b, sem_b)
        finish(0, rows_a, sem_a)
        issue_idx(b + 4, 0)
        wait_idx(b + 2, 2)
        gather_start(2, rows_a, sem_a)
        finish(1, rows_b, sem_b)
        issue_idx(b + 5, 1)
        wait_idx(b + 3, 3)
        gather_start(3, rows_b, sem_b)
        finish(2, rows_a, sem_a)
        issue_idx(b + 6, 2)
        wait_idx(b + 4, 0)
        gather_start(0, rows_a, sem_a)
        finish(3, rows_b, sem_b)
        issue_idx(b + 7, 3)
        return 0
    lax.fori_loop(0, ACH // 4 - 1, quad, 0)

    b = ACH - 4
    wait_idx(b + 1, 1)
    gather_start(1, rows_b, sem_b)
    finish(0, rows_a, sem_a)
    wait_idx(b + 2, 2)
    gather_start(2, rows_a, sem_a)
    finish(1, rows_b, sem_b)
    wait_idx(b + 3, 3)
    gather_start(3, rows_b, sem_b)
    finish(2, rows_a, sem_a)
    finish(3, rows_b, sem_b)

    plsc.subcore_barrier()

    def outcp(i, _):
        pltpu.sync_copy(acc_sh.at[pl.ds(s * RPT + i * ZR, ZR)], rows_a)
        pltpu.sync_copy(rows_a, out_hbm.at[c, pl.ds(s * RPT + i * ZR, ZR)])
        return 0
    lax.fori_loop(0, RPT // ZR, outcp, 0)


@functools.cache
def _agg_call():
    return pl.kernel(
        _agg_body,
        out_type=jax.ShapeDtypeStruct((NC, NACC, HALF), jnp.float32),
        mesh=_sc_mesh(),
        scratch_types=[
            pltpu.VMEM((4, AK), jnp.int32),
            pltpu.VMEM((4, AK), jnp.int32),
            pltpu.VMEM((AK, HALF), jnp.float32),
            pltpu.VMEM((AK, HALF), jnp.float32),
            pltpu.SemaphoreType.DMA,
            pltpu.SemaphoreType.DMA,
            pltpu.SemaphoreType.DMA,
            pltpu.SemaphoreType.DMA,
            pltpu.VMEM_SHARED((NACC, HALF), jnp.float32),
            pltpu.SemaphoreType.DMA,
            pltpu.SemaphoreType.DMA,
        ],
    )


BM = 1000
NB = N // BM


def _dinv_of(deg_ref):
    deg = deg_ref[0, :, 0:1] + deg_ref[1, :, 0:1]
    return lax.rsqrt(jnp.maximum(deg, 1.0))


def _mm0_body(x_ref, deg_ref, w_ref, out_ref):
    dinv = _dinv_of(deg_ref)
    hw = jnp.dot(x_ref[...], w_ref[...], preferred_element_type=jnp.float32)
    hs = hw * dinv
    out_ref[0] = hs[:, :HALF]
    out_ref[1] = hs[:, HALF:]


def _mm0(x, deg16, w):
    return pl.pallas_call(
        _mm0_body,
        grid=(NB,),
        in_specs=[
            pl.BlockSpec((BM, NHID), lambda i: (i, 0)),
            pl.BlockSpec((NC, BM, DEGW), lambda i: (0, i, 0)),
            pl.BlockSpec((NHID, NHID), lambda i: (0, 0)),
        ],
        out_specs=pl.BlockSpec((NC, BM, HALF), lambda i: (0, i, 0)),
        out_shape=jax.ShapeDtypeStruct((NC, N, HALF), jnp.float32),
    )(x, deg16, w)


def _stats_body(agg_ref, deg_ref, out_ref):
    i = pl.program_id(0)

    @pl.when(i == 0)
    def _():
        out_ref[...] = jnp.zeros_like(out_ref)

    dinv = _dinv_of(deg_ref)
    a = jnp.concatenate([agg_ref[0] * dinv, agg_ref[1] * dinv], axis=1)
    out_ref[0:1, :] += jnp.sum(a, axis=0, keepdims=True)
    out_ref[1:2, :] += jnp.sum(a * a, axis=0, keepdims=True)


def _stats(agg, deg16):
    return pl.pallas_call(
        _stats_body,
        grid=(NB,),
        in_specs=[
            pl.BlockSpec((NC, BM, HALF), lambda i: (0, i, 0)),
            pl.BlockSpec((NC, BM, DEGW), lambda i: (0, i, 0)),
        ],
        out_specs=pl.BlockSpec((2, NHID), lambda i: (0, 0)),
        out_shape=jax.ShapeDtypeStruct((2, NHID), jnp.float32),
    )(agg, deg16)


def _bn_h(agg_ref, deg_ref, sums_ref, gam_ref, bet_ref):
    dinv = _dinv_of(deg_ref)
    a = jnp.concatenate([agg_ref[0] * dinv, agg_ref[1] * dinv], axis=1)
    mean = sums_ref[0:1, :] * (1.0 / N)
    var = sums_ref[1:2, :] * (1.0 / N) - mean * mean
    rstd = lax.rsqrt(var + EPS)
    h = jnp.maximum((a - mean) * (rstd * gam_ref[...]) + bet_ref[...], 0.0)
    return h, dinv


def _bnmm_mid_body(agg_ref, deg_ref, sums_ref, gam_ref, bet_ref, w_ref,
                   out_ref):
    h, dinv = _bn_h(agg_ref, deg_ref, sums_ref, gam_ref, bet_ref)
    hw = jnp.dot(h * dinv, w_ref[...], preferred_element_type=jnp.float32)
    out_ref[0] = hw[:, :HALF]
    out_ref[1] = hw[:, HALF:]


def _bnmm_mid(agg, deg16, sums, gam, bet, w):
    return pl.pallas_call(
        _bnmm_mid_body,
        grid=(NB,),
        in_specs=[
            pl.BlockSpec((NC, BM, HALF), lambda i: (0, i, 0)),
            pl.BlockSpec((NC, BM, DEGW), lambda i: (0, i, 0)),
            pl.BlockSpec((2, NHID), lambda i: (0, 0)),
            pl.BlockSpec((1, NHID), lambda i: (0, 0)),
            pl.BlockSpec((1, NHID), lambda i: (0, 0)),
            pl.BlockSpec((NHID, NHID), lambda i: (0, 0)),
        ],
        out_specs=pl.BlockSpec((NC, BM, HALF), lambda i: (0, i, 0)),
        out_shape=jax.ShapeDtypeStruct((NC, N, HALF), jnp.float32),
    )(agg, deg16, sums, gam, bet, w)


def _bnmm_fin_body(agg_ref, deg_ref, sums_ref, gam_ref, bet_ref, w_ref,
                   b_ref, out_ref):
    h, _ = _bn_h(agg_ref, deg_ref, sums_ref, gam_ref, bet_ref)
    out_ref[...] = (
        jnp.dot(h, w_ref[...], preferred_element_type=jnp.float32)
        + b_ref[...]
    )


def _bnmm_fin(agg, deg16, sums, gam, bet, w, b):
    return pl.pallas_call(
        _bnmm_fin_body,
        grid=(NB,),
        in_specs=[
            pl.BlockSpec((NC, BM, HALF), lambda i: (0, i, 0)),
            pl.BlockSpec((NC, BM, DEGW), lambda i: (0, i, 0)),
            pl.BlockSpec((2, NHID), lambda i: (0, 0)),
            pl.BlockSpec((1, NHID), lambda i: (0, 0)),
            pl.BlockSpec((1, NHID), lambda i: (0, 0)),
            pl.BlockSpec((NHID, NOUT), lambda i: (0, 0)),
            pl.BlockSpec((1, NOUT), lambda i: (0, 0)),
        ],
        out_specs=pl.BlockSpec((BM, NOUT), lambda i: (i, 0)),
        out_shape=jax.ShapeDtypeStruct((N, NOUT), jnp.float32),
    )(agg, deg16, sums, gam, bet, w, b)


def kernel(x, edge_index, Wc, bc, gamma, beta, Wo, bo):
    src = edge_index[0]
    dst = edge_index[1]
    pad = EPAD - E
    srcp = jnp.concatenate([src, jnp.zeros((pad,), jnp.int32)])
    dstp = jnp.concatenate([dst, jnp.full((pad,), N, jnp.int32)])
    srcb = jnp.concatenate([srcp, srcp + N])
    deg16 = _deg_call()(dstp)
    h2 = _mm0(x, deg16, Wc[0])
    out = None
    for l in range(L):
        agg = _agg_call()(h2.reshape(NC * N, HALF), srcb, dstp)
        sums = _stats(agg, deg16)
        gam = gamma[l].reshape(1, NHID)
        bet = beta[l].reshape(1, NHID)
        if l < L - 1:
            h2 = _bnmm_mid(agg, deg16, sums, gam, bet, Wc[l + 1])
        else:
            out = _bnmm_fin(agg, deg16, sums, gam, bet, Wo,
                            bo.reshape(1, NOUT))
    return out


# speedup vs baseline: 6.2025x; 1.0152x over previous
import functools

import jax
import jax.numpy as jnp
from jax import lax
from jax.experimental import pallas as pl
from jax.experimental.pallas import tpu as pltpu
from jax.experimental.pallas import tpu_sc as plsc

N = 10000
E = 160000
NHID = 256
NOUT = 128
HALF = NHID // 2
L = 3
EPS = 1e-5

NC = 2
NS = 16
NACC = 10240
RPT = NACC // NS


EPAD = 163840
DEGW = 128
DK = 128
DET2 = EPAD // (NC * NS)
DCH2 = DET2 // DK
ZR = 128

@functools.cache
def _sc_mesh():
    return plsc.VectorSubcoreMesh(core_axis_name="c", subcore_axis_name="s")


def _deg_body(dst_hbm, deg_hbm, ones_v, zed_v, idx_v, acc_sh):
    c = lax.axis_index("c")
    s = lax.axis_index("s")
    wid = s * NC + c

    def fo(i, _):
        for j in range(DEGW // 16):
            ones_v[i, pl.ds(j * 16, 16)] = jnp.ones((16,), jnp.float32)
        return 0
    lax.fori_loop(0, DK, fo, 0)

    def fz(i, _):
        for j in range(DEGW // 16):
            zed_v[i, pl.ds(j * 16, 16)] = jnp.zeros((16,), jnp.float32)
        return 0
    lax.fori_loop(0, ZR, fz, 0)

    def zcp(i, _):
        pltpu.sync_copy(zed_v, acc_sh.at[pl.ds(s * RPT + i * ZR, ZR)])
        return 0
    lax.fori_loop(0, RPT // ZR, zcp, 0)

    plsc.subcore_barrier()

    def chunk(i, _):
        base = wid * DET2 + i * DK
        pltpu.sync_copy(dst_hbm.at[pl.ds(base, DK)], idx_v)
        pltpu.sync_copy(ones_v, acc_sh.at[idx_v], add=True)
        return 0
    lax.fori_loop(0, DCH2, chunk, 0)

    plsc.subcore_barrier()

    def outcp(i, _):
        pltpu.sync_copy(acc_sh.at[pl.ds(s * RPT + i * ZR, ZR)], zed_v)
        pltpu.sync_copy(zed_v, deg_hbm.at[c, pl.ds(s * RPT + i * ZR, ZR)])
        return 0
    lax.fori_loop(0, RPT // ZR, outcp, 0)


@functools.cache
def _deg_call():
    return pl.kernel(
        _deg_body,
        out_type=jax.ShapeDtypeStruct((NC, NACC, DEGW), jnp.float32),
        mesh=_sc_mesh(),
        scratch_types=[
            pltpu.VMEM((DK, DEGW), jnp.float32),
            pltpu.VMEM((ZR, DEGW), jnp.float32),
            pltpu.VMEM((DK,), jnp.int32),
            pltpu.VMEM_SHARED((NACC, DEGW), jnp.float32),
        ],
    )


AK = 64
AET = EPAD // NS
ACH = AET // AK


def _agg_body(hw2_hbm, srcb_hbm, dst_hbm, out_hbm,
              idxs_v, idxd_v, rows_0, rows_1, rows_2, rows_3,
              sem_i0, sem_i1, sem_i2, sem_i3, sem_i4, sem_i5, sem_i6, sem_i7,
              acc_sh, sem_g0, sem_g1, sem_g2, sem_g3):
    c = lax.axis_index("c")
    s = lax.axis_index("s")
    sb = c * EPAD + s * AET
    db = s * AET
    sem_i = (sem_i0, sem_i1, sem_i2, sem_i3, sem_i4, sem_i5, sem_i6, sem_i7)
    rows = (rows_0, rows_1, rows_2, rows_3)
    sem_g = (sem_g0, sem_g1, sem_g2, sem_g3)

    def issue_idx(q, j):
        pltpu.async_copy(srcb_hbm.at[pl.ds(sb + q * AK, AK)],
                         idxs_v.at[j], sem_i[j])
        pltpu.async_copy(dst_hbm.at[pl.ds(db + q * AK, AK)],
                         idxd_v.at[j], sem_i[j])

    def wait_idx(q, j):
        pltpu.make_async_copy(srcb_hbm.at[pl.ds(sb + q * AK, AK)],
                              idxs_v.at[j], sem_i[j]).wait()
        pltpu.make_async_copy(dst_hbm.at[pl.ds(db + q * AK, AK)],
                              idxd_v.at[j], sem_i[j]).wait()

    def gather_start(j, r):
        pltpu.async_copy(hw2_hbm.at[idxs_v.at[j]], rows[r], sem_g[r])

    def finish(j, r):
        pltpu.make_async_copy(hw2_hbm.at[idxs_v.at[j]], rows[r],
                              sem_g[r]).wait()
        pltpu.sync_copy(rows[r], acc_sh.at[idxd_v.at[j]], add=True)

    for j in range(8):
        issue_idx(j, j)

    def fz(i, _):
        for j in range(HALF // 16):
            rows_0[i, pl.ds(j * 16, 16)] = jnp.zeros((16,), jnp.float32)
        return 0
    lax.fori_loop(0, AK, fz, 0)

    def zcp(i, _):
        pltpu.sync_copy(rows_0, acc_sh.at[pl.ds(s * RPT + i * AK, AK)])
        return 0
    lax.fori_loop(0, RPT // AK, zcp, 0)

    for q in range(4):
        wait_idx(q, q)
        gather_start(q, q)
    plsc.subcore_barrier()

    def octet(t, _):
        b = 8 * t
        for j in range(8):
            q = b + j
            finish(j, j % 4)
            issue_idx(q + 8, j)
            wait_idx(q + 4, (j + 4) % 8)
            gather_start((j + 4) % 8, j % 4)
        return 0
    lax.fori_loop(0, ACH // 8 - 1, octet, 0)

    b = ACH - 8
    for j in range(8):
        q = b + j
        finish(j, j % 4)
        if j < 4:
            wait_idx(q + 4, (j + 4) % 8)
            gather_start((j + 4) % 8, j % 4)

    plsc.subcore_barrier()

    def outcp(i, _):
        pltpu.sync_copy(acc_sh.at[pl.ds(s * RPT + i * AK, AK)], rows_0)
        pltpu.sync_copy(rows_0, out_hbm.at[c, pl.ds(s * RPT + i * AK, AK)])
        return 0
    lax.fori_loop(0, RPT // AK, outcp, 0)


@functools.cache
def _agg_call():
    return pl.kernel(
        _agg_body,
        out_type=jax.ShapeDtypeStruct((NC, NACC, HALF), jnp.float32),
        mesh=_sc_mesh(),
        scratch_types=[
            pltpu.VMEM((8, AK), jnp.int32),
            pltpu.VMEM((8, AK), jnp.int32),
            pltpu.VMEM((AK, HALF), jnp.float32),
            pltpu.VMEM((AK, HALF), jnp.float32),
            pltpu.VMEM((AK, HALF), jnp.float32),
            pltpu.VMEM((AK, HALF), jnp.float32),
            pltpu.SemaphoreType.DMA,
            pltpu.SemaphoreType.DMA,
            pltpu.SemaphoreType.DMA,
            pltpu.SemaphoreType.DMA,
            pltpu.SemaphoreType.DMA,
            pltpu.SemaphoreType.DMA,
            pltpu.SemaphoreType.DMA,
            pltpu.SemaphoreType.DMA,
            pltpu.VMEM_SHARED((NACC, HALF), jnp.float32),
            pltpu.SemaphoreType.DMA,
            pltpu.SemaphoreType.DMA,
            pltpu.SemaphoreType.DMA,
            pltpu.SemaphoreType.DMA,
        ],
    )


BM = 1000
NB = N // BM


def _dinv_of(deg_ref):
    deg = deg_ref[0, :, 0:1] + deg_ref[1, :, 0:1]
    return lax.rsqrt(jnp.maximum(deg, 1.0))


def _mm0_body(x_ref, deg_ref, w_ref, out_ref):
    dinv = _dinv_of(deg_ref)
    hw = jnp.dot(x_ref[...], w_ref[...], preferred_element_type=jnp.float32)
    hs = hw * dinv
    out_ref[0] = hs[:, :HALF]
    out_ref[1] = hs[:, HALF:]


def _mm0(x, deg16, w):
    return pl.pallas_call(
        _mm0_body,
        grid=(NB,),
        in_specs=[
            pl.BlockSpec((BM, NHID), lambda i: (i, 0)),
            pl.BlockSpec((NC, BM, DEGW), lambda i: (0, i, 0)),
            pl.BlockSpec((NHID, NHID), lambda i: (0, 0)),
        ],
        out_specs=pl.BlockSpec((NC, BM, HALF), lambda i: (0, i, 0)),
        out_shape=jax.ShapeDtypeStruct((NC, N, HALF), jnp.float32),
    )(x, deg16, w)


def _stats_body(agg_ref, deg_ref, out_ref):
    i = pl.program_id(0)

    @pl.when(i == 0)
    def _():
        out_ref[...] = jnp.zeros_like(out_ref)

    dinv = _dinv_of(deg_ref)
    a = jnp.concatenate([agg_ref[0] * dinv, agg_ref[1] * dinv], axis=1)
    out_ref[0:1, :] += jnp.sum(a, axis=0, keepdims=True)
    out_ref[1:2, :] += jnp.sum(a * a, axis=0, keepdims=True)


def _stats(agg, deg16):
    return pl.pallas_call(
        _stats_body,
        grid=(NB,),
        in_specs=[
            pl.BlockSpec((NC, BM, HALF), lambda i: (0, i, 0)),
            pl.BlockSpec((NC, BM, DEGW), lambda i: (0, i, 0)),
        ],
        out_specs=pl.BlockSpec((2, NHID), lambda i: (0, 0)),
        out_shape=jax.ShapeDtypeStruct((2, NHID), jnp.float32),
    )(agg, deg16)


def _bn_h(agg_ref, deg_ref, sums_ref, gam_ref, bet_ref):
    dinv = _dinv_of(deg_ref)
    a = jnp.concatenate([agg_ref[0] * dinv, agg_ref[1] * dinv], axis=1)
    mean = sums_ref[0:1, :] * (1.0 / N)
    var = sums_ref[1:2, :] * (1.0 / N) - mean * mean
    rstd = lax.rsqrt(var + EPS)
    h = jnp.maximum((a - mean) * (rstd * gam_ref[...]) + bet_ref[...], 0.0)
    return h, dinv


def _bnmm_mid_body(agg_ref, deg_ref, sums_ref, gam_ref, bet_ref, w_ref,
                   out_ref):
    h, dinv = _bn_h(agg_ref, deg_ref, sums_ref, gam_ref, bet_ref)
    hw = jnp.dot(h * dinv, w_ref[...], preferred_element_type=jnp.float32)
    out_ref[0] = hw[:, :HALF]
    out_ref[1] = hw[:, HALF:]


def _bnmm_mid(agg, deg16, sums, gam, bet, w):
    return pl.pallas_call(
        _bnmm_mid_body,
        grid=(NB,),
        in_specs=[
            pl.BlockSpec((NC, BM, HALF), lambda i: (0, i, 0)),
            pl.BlockSpec((NC, BM, DEGW), lambda i: (0, i, 0)),
            pl.BlockSpec((2, NHID), lambda i: (0, 0)),
            pl.BlockSpec((1, NHID), lambda i: (0, 0)),
            pl.BlockSpec((1, NHID), lambda i: (0, 0)),
            pl.BlockSpec((NHID, NHID), lambda i: (0, 0)),
        ],
        out_specs=pl.BlockSpec((NC, BM, HALF), lambda i: (0, i, 0)),
        out_shape=jax.ShapeDtypeStruct((NC, N, HALF), jnp.float32),
    )(agg, deg16, sums, gam, bet, w)


def _bnmm_fin_body(agg_ref, deg_ref, sums_ref, gam_ref, bet_ref, w_ref,
                   b_ref, out_ref):
    h, _ = _bn_h(agg_ref, deg_ref, sums_ref, gam_ref, bet_ref)
    out_ref[...] = (
        jnp.dot(h, w_ref[...], preferred_element_type=jnp.float32)
        + b_ref[...]
    )


def _bnmm_fin(agg, deg16, sums, gam, bet, w, b):
    return pl.pallas_call(
        _bnmm_fin_body,
        grid=(NB,),
        in_specs=[
            pl.BlockSpec((NC, BM, HALF), lambda i: (0, i, 0)),
            pl.BlockSpec((NC, BM, DEGW), lambda i: (0, i, 0)),
            pl.BlockSpec((2, NHID), lambda i: (0, 0)),
            pl.BlockSpec((1, NHID), lambda i: (0, 0)),
            pl.BlockSpec((1, NHID), lambda i: (0, 0)),
            pl.BlockSpec((NHID, NOUT), lambda i: (0, 0)),
            pl.BlockSpec((1, NOUT), lambda i: (0, 0)),
        ],
        out_specs=pl.BlockSpec((BM, NOUT), lambda i: (i, 0)),
        out_shape=jax.ShapeDtypeStruct((N, NOUT), jnp.float32),
    )(agg, deg16, sums, gam, bet, w, b)


def kernel(x, edge_index, Wc, bc, gamma, beta, Wo, bo):
    src = edge_index[0]
    dst = edge_index[1]
    pad = EPAD - E
    srcp = jnp.concatenate([src, jnp.zeros((pad,), jnp.int32)])
    dstp = jnp.concatenate([dst, jnp.full((pad,), N, jnp.int32)])
    srcb = jnp.concatenate([srcp, srcp + N])
    deg16 = _deg_call()(dstp)
    h2 = _mm0(x, deg16, Wc[0])
    out = None
    for l in range(L):
        agg = _agg_call()(h2.reshape(NC * N, HALF), srcb, dstp)
        sums = _stats(agg, deg16)
        gam = gamma[l].reshape(1, NHID)
        bet = beta[l].reshape(1, NHID)
        if l < L - 1:
            h2 = _bnmm_mid(agg, deg16, sums, gam, bet, Wc[l + 1])
        else:
            out = _bnmm_fin(agg, deg16, sums, gam, bet, Wo,
                            bo.reshape(1, NOUT))
    return out
